# baseline (speedup 1.0000x reference)
import functools

import jax
import jax.numpy as jnp
from jax import lax
from jax.experimental import pallas as pl
from jax.experimental.pallas import tpu as pltpu
from jax.experimental.pallas import tpu_sc as plsc

_NBUF = 4
_NCHUNK = 8
_LOOKAHEAD = 2


@functools.lru_cache(maxsize=None)
def _build(batch: int, n_rows: int, dim: int):
    info = plsc.get_sparse_core_info()
    nc, ns = info.num_cores, info.num_subcores
    nw = nc * ns
    assert batch % (8 * nw * _NCHUNK) == 0, (batch, nw)
    b_per_w = batch // nw
    cb = b_per_w // _NCHUNK
    mesh = plsc.VectorSubcoreMesh(core_axis_name="c", subcore_axis_name="s")

    @functools.partial(
        pl.kernel,
        mesh=mesh,
        out_type=jax.ShapeDtypeStruct((batch, dim), jnp.float32),
        scratch_types=[
            pltpu.VMEM((_NCHUNK, cb), jnp.int32),
        ]
        + [pltpu.VMEM((cb, dim), jnp.float32) for _ in range(_NBUF)]
        + [pltpu.SemaphoreType.DMA for _ in range(2 * _NBUF)],
    )
    def k(idx_hbm, table_hbm, out_hbm, idx_v, *rest):
        bufs = rest[:_NBUF]
        gsems = rest[_NBUF : 2 * _NBUF]
        osems = rest[2 * _NBUF :]
        wid = lax.axis_index("s") * nc + lax.axis_index("c")
        base = wid * b_per_w
        pltpu.sync_copy(idx_hbm.at[wid], idx_v)

        gcp = [None] * _NBUF
        ocp = [None] * _NBUF

        def gather(g):
            b = g % _NBUF
            return pltpu.async_copy(table_hbm.at[idx_v.at[g]], bufs[b], gsems[b])

        for j in range(min(_LOOKAHEAD, _NCHUNK)):
            gcp[j % _NBUF] = gather(j)
        for g in range(_NCHUNK):
            b = g % _NBUF
            nxt = g + _LOOKAHEAD
            if nxt < _NCHUNK:
                nb = nxt % _NBUF
                if ocp[nb] is not None:
                    ocp[nb].wait()
                    ocp[nb] = None
                gcp[nb] = gather(nxt)
            gcp[b].wait()
            ocp[b] = pltpu.async_copy(
                bufs[b], out_hbm.at[pl.ds(base + g * cb, cb)], osems[b]
            )
        for b in range(_NBUF):
            if ocp[b] is not None:
                ocp[b].wait()

    return k


def kernel(gene_ids, embedding_weight):
    (batch,) = gene_ids.shape
    n_rows, dim = embedding_weight.shape
    k = _build(batch, n_rows, dim)
    info = plsc.get_sparse_core_info()
    nw = info.num_cores * info.num_subcores
    cb = batch // (nw * _NCHUNK)
    idx = gene_ids.astype(jnp.int32).reshape(nw, _NCHUNK, cb)
    return k(idx, embedding_weight)


# speedup vs baseline: 1.0503x; 1.0503x over previous
import functools

import jax
import jax.numpy as jnp
from jax import lax
from jax.experimental import pallas as pl
from jax.experimental.pallas import tpu as pltpu
from jax.experimental.pallas import tpu_sc as plsc


@functools.lru_cache(maxsize=None)
def _build(batch: int, n_rows: int, dim: int):
    info = plsc.get_sparse_core_info()
    nc, ns = info.num_cores, info.num_subcores
    nw = nc * ns
    assert batch % (8 * nw) == 0, (batch, nw)
    b_per_w = batch // nw
    mesh = plsc.VectorSubcoreMesh(core_axis_name="c", subcore_axis_name="s")

    @functools.partial(
        pl.kernel,
        mesh=mesh,
        out_type=jax.ShapeDtypeStruct((batch, dim), jnp.float32),
        scratch_types=[
            pltpu.VMEM((b_per_w,), jnp.int32),
            pltpu.VMEM((b_per_w, dim), jnp.float32),
            pltpu.SemaphoreType.DMA,
        ],
    )
    def k(idx_hbm, table_hbm, out_hbm, idx_v, rows_v, sem):
        wid = lax.axis_index("s") * nc + lax.axis_index("c")
        base = wid * b_per_w
        pltpu.sync_copy(idx_hbm.at[pl.ds(base, b_per_w)], idx_v)
        pltpu.async_copy(table_hbm.at[idx_v], rows_v, sem).wait()
        pltpu.sync_copy(rows_v, out_hbm.at[pl.ds(base, b_per_w)])

    return k


def kernel(gene_ids, embedding_weight):
    (batch,) = gene_ids.shape
    n_rows, dim = embedding_weight.shape
    k = _build(batch, n_rows, dim)
    return k(gene_ids.astype(jnp.int32), embedding_weight)
